# Initial kernel scaffold; baseline (speedup 1.0000x reference)
import jax
import jax.numpy as jnp
from jax.experimental import pallas as pl


def kernel(atomic_energies, atomic_numbers, scale, shift):
    raise NotImplementedError("write your pallas kernel here")


# speedup vs baseline: 645.4686x; 645.4686x over previous
import functools

import jax
import jax.numpy as jnp
from jax import lax
from jax.experimental import pallas as pl
from jax.experimental.pallas import tpu as pltpu
from jax.experimental.pallas import tpu_sc as plsc

N = 1600000
NUM_CORES = 2
NUM_SUBCORES = 16
NUM_WORKERS = NUM_CORES * NUM_SUBCORES
CHUNK = N // NUM_WORKERS
LANES = 16
TBL_PAD = 96

_mesh = plsc.VectorSubcoreMesh(core_axis_name="c", subcore_axis_name="s")


@functools.partial(
    pl.kernel,
    mesh=_mesh,
    out_type=jax.ShapeDtypeStruct((N,), jnp.float32),
    scratch_types=[
        pltpu.VMEM((CHUNK,), jnp.float32),
        pltpu.VMEM((CHUNK,), jnp.int32),
        pltpu.VMEM((TBL_PAD,), jnp.float32),
        pltpu.VMEM((TBL_PAD,), jnp.float32),
        pltpu.SemaphoreType.DMA,
    ],
    compiler_params=pltpu.CompilerParams(needs_layout_passes=False),
)
def _atom_scale_sc(e_hbm, z_hbm, scale_hbm, shift_hbm, out_hbm,
                   e_v, z_v, scale_v, shift_v, sem):
    wid = lax.axis_index("s") * NUM_CORES + lax.axis_index("c")
    base = wid * CHUNK
    d0 = pltpu.async_copy(scale_hbm, scale_v, sem)
    d1 = pltpu.async_copy(shift_hbm, shift_v, sem)
    d2 = pltpu.async_copy(e_hbm.at[pl.ds(base, CHUNK)], e_v, sem)
    d3 = pltpu.async_copy(z_hbm.at[pl.ds(base, CHUNK)], z_v, sem)
    d0.wait()
    d1.wait()
    d2.wait()
    d3.wait()

    @plsc.parallel_loop(0, CHUNK // LANES, unroll=8)
    def body(i):
        off = i * LANES
        idx = z_v[pl.ds(off, LANES)]
        s = plsc.load_gather(scale_v, [idx])
        t = plsc.load_gather(shift_v, [idx])
        e_v[pl.ds(off, LANES)] = s * e_v[pl.ds(off, LANES)] + t

    pltpu.sync_copy(e_v, out_hbm.at[pl.ds(base, CHUNK)])


def kernel(atomic_energies, atomic_numbers, scale, shift):
    scale_p = jnp.pad(scale, (0, TBL_PAD - scale.shape[0]))
    shift_p = jnp.pad(shift, (0, TBL_PAD - shift.shape[0]))
    return _atom_scale_sc(atomic_energies, atomic_numbers, scale_p, shift_p)


# speedup vs baseline: 693.2432x; 1.0740x over previous
import functools

import jax
import jax.numpy as jnp
from jax import lax
from jax.experimental import pallas as pl
from jax.experimental.pallas import tpu as pltpu
from jax.experimental.pallas import tpu_sc as plsc

N = 1600000
NUM_CORES = 2
NUM_SUBCORES = 16
NUM_WORKERS = NUM_CORES * NUM_SUBCORES
CHUNK = N // NUM_WORKERS
K = 5
TILE = CHUNK // K
LANES = 16
TBL_PAD = 96

_mesh = plsc.VectorSubcoreMesh(core_axis_name="c", subcore_axis_name="s")


@functools.partial(
    pl.kernel,
    mesh=_mesh,
    out_type=jax.ShapeDtypeStruct((N,), jnp.float32),
    scratch_types=[
        pltpu.VMEM((TILE,), jnp.float32),
        pltpu.VMEM((TILE,), jnp.float32),
        pltpu.VMEM((TILE,), jnp.int32),
        pltpu.VMEM((TILE,), jnp.int32),
        pltpu.VMEM((TBL_PAD,), jnp.float32),
        pltpu.VMEM((TBL_PAD,), jnp.float32),
        pltpu.SemaphoreType.DMA,
        pltpu.SemaphoreType.DMA,
        pltpu.SemaphoreType.DMA,
        pltpu.SemaphoreType.DMA,
    ],
    compiler_params=pltpu.CompilerParams(needs_layout_passes=False),
)
def _atom_scale_sc(e_hbm, z_hbm, scale_hbm, shift_hbm, out_hbm,
                   e0, e1, z0, z1, scale_v, shift_v,
                   sem_in0, sem_in1, sem_out0, sem_out1):
    wid = lax.axis_index("s") * NUM_CORES + lax.axis_index("c")
    base = wid * CHUNK

    e_bufs = (e0, e1)
    z_bufs = (z0, z1)
    sems_in = (sem_in0, sem_in1)
    sems_out = (sem_out0, sem_out1)

    t0 = pltpu.async_copy(scale_hbm, scale_v, sems_in[1])
    t1 = pltpu.async_copy(shift_hbm, shift_v, sems_in[1])

    def fire_in(k):
        b = k & 1
        off = base + k * TILE
        he = pltpu.async_copy(e_hbm.at[pl.ds(off, TILE)], e_bufs[b], sems_in[b])
        hz = pltpu.async_copy(z_hbm.at[pl.ds(off, TILE)], z_bufs[b], sems_in[b])
        return he, hz

    in_h = [None] * K
    in_h[0] = fire_in(0)
    t0.wait()
    t1.wait()

    out_h = [None] * K
    for k in range(K):
        b = k & 1
        if k + 1 < K:
            if k >= 1:
                out_h[k - 1][0].wait()
            in_h[k + 1] = fire_in(k + 1)
        in_h[k][0].wait()
        in_h[k][1].wait()
        ev = e_bufs[b]
        zv = z_bufs[b]

        @plsc.parallel_loop(0, TILE // LANES, unroll=8)
        def body(i):
            off = i * LANES
            idx = zv[pl.ds(off, LANES)]
            s = plsc.load_gather(scale_v, [idx])
            t = plsc.load_gather(shift_v, [idx])
            ev[pl.ds(off, LANES)] = s * ev[pl.ds(off, LANES)] + t

        out_h[k] = (
            pltpu.async_copy(
                ev, out_hbm.at[pl.ds(base + k * TILE, TILE)], sems_out[b]
            ),
        )

    out_h[K - 2][0].wait()
    out_h[K - 1][0].wait()


def kernel(atomic_energies, atomic_numbers, scale, shift):
    scale_p = jnp.pad(scale, (0, TBL_PAD - scale.shape[0]))
    shift_p = jnp.pad(shift, (0, TBL_PAD - shift.shape[0]))
    return _atom_scale_sc(atomic_energies, atomic_numbers, scale_p, shift_p)
